# Initial kernel scaffold; baseline (speedup 1.0000x reference)
import jax
import jax.numpy as jnp
from jax.experimental import pallas as pl


def kernel(x, edge_index, W, a):
    raise NotImplementedError("write your pallas kernel here")


# speedup vs baseline: 16.5966x; 16.5966x over previous
import functools

import jax
import jax.numpy as jnp
from jax import lax
from jax.experimental import pallas as pl
from jax.experimental.pallas import tpu as pltpu
from jax.experimental.pallas import tpu_sc as plsc

N = 10000
E = 320000
D = 128
ALPHA = 0.2

NC, NS, L = 2, 16, 16
NW = NC * NS
NPAD = 10240
ROWS_PER_TILE = NPAD // NS
EPW = E // NW
CHUNK = 80
NCHUNKS = EPW // CHUNK


def _tc_prep(x_ref, w_ref, a_ref, h_ref, s1_ref, s2_ref, c_ref):
    h = jnp.dot(x_ref[...], w_ref[...], preferred_element_type=jnp.float32)
    h_ref[...] = h
    a = a_ref[...]
    s1 = jnp.sum(h * a[:D, 0][None, :], axis=1)
    s2 = jnp.sum(h * a[D:, 0][None, :], axis=1)
    s1_ref[...] = s1
    s2_ref[...] = s2
    m = jnp.max(s1) + jnp.max(s2)
    c_ref[...] = jnp.full((16,), jnp.where(m >= 0.0, m, ALPHA * m),
                          dtype=jnp.float32)


def _sc_edges(h_hbm, row_hbm, col_hbm, s1_hbm, s2_hbm, c_hbm,
              znd_hbm, zn_hbm, u_out, rs_out,
              s1_v, s2_v, c_v, row_v, col_v, rows_v, w_v,
              u_sh, rs_sh, sem):
    cid = lax.axis_index("c")
    sid = lax.axis_index("s")
    wid = cid * NS + sid
    pltpu.sync_copy(s1_hbm, s1_v)
    pltpu.sync_copy(s2_hbm, s2_v)
    pltpu.sync_copy(c_hbm, c_v)
    r0 = sid * ROWS_PER_TILE
    pltpu.sync_copy(znd_hbm.at[pl.ds(r0, ROWS_PER_TILE)],
                    u_sh.at[pl.ds(r0, ROWS_PER_TILE)])
    pltpu.sync_copy(zn_hbm.at[pl.ds(r0, ROWS_PER_TILE)],
                    rs_sh.at[pl.ds(r0, ROWS_PER_TILE)])
    plsc.subcore_barrier()

    cvec = c_v[pl.ds(0, L)]
    base = wid * EPW

    def chunk_body(g, carry):
        off = base + g * CHUNK
        pltpu.sync_copy(row_hbm.at[pl.ds(off, CHUNK)], row_v)
        pltpu.sync_copy(col_hbm.at[pl.ds(off, CHUNK)], col_v)
        pltpu.async_copy(h_hbm.at[col_v], rows_v, sem).wait()
        for i in range(CHUNK // L):
            idxr = row_v[pl.ds(i * L, L)]
            idxc = col_v[pl.ds(i * L, L)]
            e = plsc.load_gather(s1_v, [idxr]) + plsc.load_gather(s2_v, [idxc])
            e = jnp.where(e >= 0.0, e, ALPHA * e)
            w_v[pl.ds(i * L, L)] = jnp.exp(e - cvec)

        def scale_body(ei, c2):
            ws = plsc.load_gather(w_v, [jnp.full((L,), ei, jnp.int32)])
            for j in range(D // L):
                rows_v[ei, pl.ds(j * L, L)] = rows_v[ei, pl.ds(j * L, L)] * ws
            return c2

        lax.fori_loop(0, CHUNK, scale_body, 0)
        pltpu.sync_copy(rows_v, u_sh.at[row_v], add=True)
        pltpu.sync_copy(w_v, rs_sh.at[row_v], add=True)
        return carry

    lax.fori_loop(0, NCHUNKS, chunk_body, 0)
    plsc.subcore_barrier()
    pltpu.sync_copy(u_sh.at[pl.ds(r0, ROWS_PER_TILE)],
                    u_out.at[cid, pl.ds(r0, ROWS_PER_TILE)])
    pltpu.sync_copy(rs_sh.at[pl.ds(r0, ROWS_PER_TILE)],
                    rs_out.at[cid, pl.ds(r0, ROWS_PER_TILE)])


_sc_edges_call = functools.partial(
    pl.kernel,
    out_type=[jax.ShapeDtypeStruct((NC, NPAD, D), jnp.float32),
              jax.ShapeDtypeStruct((NC, NPAD), jnp.float32)],
    mesh=plsc.VectorSubcoreMesh(core_axis_name="c", subcore_axis_name="s"),
    compiler_params=pltpu.CompilerParams(needs_layout_passes=False),
    scratch_types=[
        pltpu.VMEM((N,), jnp.float32),
        pltpu.VMEM((N,), jnp.float32),
        pltpu.VMEM((16,), jnp.float32),
        pltpu.VMEM((CHUNK,), jnp.int32),
        pltpu.VMEM((CHUNK,), jnp.int32),
        pltpu.VMEM((CHUNK, D), jnp.float32),
        pltpu.VMEM((CHUNK,), jnp.float32),
        pltpu.VMEM_SHARED((NPAD, D), jnp.float32),
        pltpu.VMEM_SHARED((NPAD,), jnp.float32),
        pltpu.SemaphoreType.DMA,
    ],
)(_sc_edges)


def _tc_final(u_ref, rs_ref, o_ref):
    u = u_ref[0] + u_ref[1]
    rs = jnp.clip(rs_ref[0] + rs_ref[1], 1e-8, None)
    hp = u / rs[:, None]
    o_ref[...] = jnp.where(hp > 0.0, hp, jnp.exp(jnp.minimum(hp, 0.0)) - 1.0)


def kernel(x, edge_index, W, a):
    h, s1, s2, c = pl.pallas_call(
        _tc_prep,
        out_shape=[
            jax.ShapeDtypeStruct((N, D), jnp.float32),
            jax.ShapeDtypeStruct((N,), jnp.float32),
            jax.ShapeDtypeStruct((N,), jnp.float32),
            jax.ShapeDtypeStruct((16,), jnp.float32),
        ],
    )(x, W, a)
    row = edge_index[0]
    col = edge_index[1]
    znd = jnp.zeros((NPAD, D), jnp.float32)
    zn = jnp.zeros((NPAD,), jnp.float32)
    u_parts, rs_parts = _sc_edges_call(h, row, col, s1, s2, c, znd, zn)
    out = pl.pallas_call(
        _tc_final,
        out_shape=jax.ShapeDtypeStruct((NPAD, D), jnp.float32),
    )(u_parts, rs_parts)
    return out[:N]
